# Initial kernel scaffold; baseline (speedup 1.0000x reference)
import jax
import jax.numpy as jnp
from jax.experimental import pallas as pl


def kernel(f_atoms, f_bonds, a2b, b2a, b2revb, n_mols, a_size, W_i_atom, W_i_bond, W_h, W_lr, W_o, b_o, gru_bias, Wih_f, Whh_f, bih_f, bhh_f, Wih_b, Whh_b, bih_b, bhh_b):
    raise NotImplementedError("write your pallas kernel here")


# speedup vs baseline: 1.2624x; 1.2624x over previous
import functools

import jax
import jax.numpy as jnp
from jax import lax
from jax.experimental import pallas as pl
from jax.experimental.pallas import tpu as pltpu
from jax.experimental.pallas import tpu_sc as plsc

H = 512
DEPTH = 6
N_MOLS = 200
A_SIZE = 50
N_ATOMS = 1 + N_MOLS * A_SIZE
N_BONDS = 20001
MAX_NB = 6
NAP = 10240
NBP = 20480
FPAD = 256

NW = 32
_F32 = jnp.float32


def _make_agg_kernel(include_atom: bool):
    APW = NAP // NW
    G = 8
    G6 = G * MAX_NB
    mesh = plsc.VectorSubcoreMesh(core_axis_name="c", subcore_axis_name="s")

    @functools.partial(
        pl.kernel, mesh=mesh,
        out_type=jax.ShapeDtypeStruct((NAP, H), _F32),
        scratch_types=[
            pltpu.VMEM((G6,), jnp.int32),
            pltpu.VMEM((G6, H), _F32),
            pltpu.VMEM((G, H), _F32),
            pltpu.SemaphoreType.DMA,
        ],
    )
    def k(bonds, atoms, idx, out, idx_v, rows_v, acc_v, sem):
        wid = lax.axis_index("s") * 2 + lax.axis_index("c")
        base = wid * APW

        def batch(i, carry):
            a0 = base + i * G
            pltpu.sync_copy(idx.at[pl.ds(a0 * MAX_NB, G6)], idx_v)
            pltpu.async_copy(bonds.at[idx_v], rows_v, sem).wait()
            if include_atom:
                pltpu.sync_copy(atoms.at[pl.ds(a0, G)], acc_v)

            def per_g(g, c2):
                def per_c(c, c3):
                    col = pl.ds(c * 16, 16)
                    s = rows_v[g * MAX_NB, col]
                    for kk in range(1, MAX_NB):
                        s = s + rows_v[g * MAX_NB + kk, col]
                    s = s * (1.0 / MAX_NB)
                    if include_atom:
                        s = s + acc_v[g, col]
                    acc_v[g, col] = s
                    return c3
                return lax.fori_loop(0, H // 16, per_c, c2)

            lax.fori_loop(0, G, per_g, 0)
            pltpu.sync_copy(acc_v, out.at[pl.ds(a0, G)])
            return carry

        lax.fori_loop(0, APW // G, batch, 0)

    return k


def _make_bondpre_kernel():
    BPW = NBP // NW
    G = 16
    mesh = plsc.VectorSubcoreMesh(core_axis_name="c", subcore_axis_name="s")

    @functools.partial(
        pl.kernel, mesh=mesh,
        out_type=jax.ShapeDtypeStruct((NBP, H), _F32),
        scratch_types=[
            pltpu.VMEM((G,), jnp.int32),
            pltpu.VMEM((G,), jnp.int32),
            pltpu.VMEM((G, H), _F32),
            pltpu.VMEM((G, H), _F32),
            pltpu.SemaphoreType.DMA,
            pltpu.SemaphoreType.DMA,
        ],
    )
    def k(atoms, bonds, b2a, b2revb, out, ia_v, ib_v, ra_v, rb_v, sa, sb):
        wid = lax.axis_index("s") * 2 + lax.axis_index("c")
        base = wid * BPW

        def batch(i, carry):
            b0 = base + i * G
            pltpu.sync_copy(b2a.at[pl.ds(b0, G)], ia_v)
            pltpu.sync_copy(b2revb.at[pl.ds(b0, G)], ib_v)
            ca = pltpu.async_copy(atoms.at[ia_v], ra_v, sa)
            cb = pltpu.async_copy(bonds.at[ib_v], rb_v, sb)
            ca.wait()
            cb.wait()

            def per_g(g, c2):
                def per_c(c, c3):
                    col = pl.ds(c * 16, 16)
                    ra_v[g, col] = ra_v[g, col] - rb_v[g, col]
                    return c3
                return lax.fori_loop(0, H // 16, per_c, c2)

            lax.fori_loop(0, G, per_g, 0)
            pltpu.sync_copy(ra_v, out.at[pl.ds(b0, G)])
            return carry

        lax.fori_loop(0, BPW // G, batch, 0)

    return k


_agg_with_atom = _make_agg_kernel(True)
_agg_only = _make_agg_kernel(False)
_bondpre = _make_bondpre_kernel()


def _proj_relu(x, w, rows_blk):
    R, K = x.shape
    N = w.shape[1]

    def body(x_ref, w_ref, o_ref):
        o_ref[...] = jnp.maximum(
            jnp.dot(x_ref[...], w_ref[...], preferred_element_type=_F32), 0.0)

    return pl.pallas_call(
        body,
        grid=(R // rows_blk,),
        in_specs=[
            pl.BlockSpec((rows_blk, K), lambda i: (i, 0)),
            pl.BlockSpec((K, N), lambda i: (0, 0)),
        ],
        out_specs=pl.BlockSpec((rows_blk, N), lambda i: (i, 0)),
        out_shape=jax.ShapeDtypeStruct((R, N), _F32),
    )(x, w)


def _bond_update(pre, inb, w, rows_blk=1024):
    R = pre.shape[0]

    def body(p_ref, b_ref, w_ref, o_ref):
        o_ref[...] = jnp.maximum(
            b_ref[...] + jnp.dot(p_ref[...], w_ref[...],
                                 preferred_element_type=_F32), 0.0)

    return pl.pallas_call(
        body,
        grid=(R // rows_blk,),
        in_specs=[
            pl.BlockSpec((rows_blk, H), lambda i: (i, 0)),
            pl.BlockSpec((rows_blk, H), lambda i: (i, 0)),
            pl.BlockSpec((H, H), lambda i: (0, 0)),
        ],
        out_specs=pl.BlockSpec((rows_blk, H), lambda i: (i, 0)),
        out_shape=jax.ShapeDtypeStruct((R, H), _F32),
    )(pre, inb, w)


def _lr_combine(agg, matom, iatom, w_lr, rows_blk=1024):
    R = agg.shape[0]

    def body(a_ref, m_ref, i_ref, w_ref, o_ref):
        w = w_ref[...]
        o_ref[...] = (
            jnp.dot(a_ref[...], w[0:H], preferred_element_type=_F32)
            + jnp.dot(m_ref[...], w[H:2 * H], preferred_element_type=_F32)
            + jnp.dot(i_ref[...], w[2 * H:3 * H], preferred_element_type=_F32))

    return pl.pallas_call(
        body,
        grid=(R // rows_blk,),
        in_specs=[
            pl.BlockSpec((rows_blk, H), lambda i: (i, 0)),
            pl.BlockSpec((rows_blk, H), lambda i: (i, 0)),
            pl.BlockSpec((rows_blk, H), lambda i: (i, 0)),
            pl.BlockSpec((3 * H, H), lambda i: (0, 0)),
        ],
        out_specs=pl.BlockSpec((rows_blk, H), lambda i: (i, 0)),
        out_shape=jax.ShapeDtypeStruct((R, H), _F32),
    )(agg, matom, iatom, w_lr)


def _gru_pre(node_tm, wih_f, bih_f, wih_b, bih_b, gbias):
    TB = 2
    T = A_SIZE
    M = N_MOLS

    def body(x_ref, wf_ref, bf_ref, wb_ref, bb_ref, gb_ref,
             gif_ref, gib_ref, hp_ref):
        i = pl.program_id(0)
        x = x_ref[...]
        x2 = x.reshape(TB * M, H)
        msg = jnp.maximum(x2 + gb_ref[...], 0.0)
        gif = jnp.dot(msg, wf_ref[...], preferred_element_type=_F32) + bf_ref[...]
        gib = jnp.dot(msg, wb_ref[...], preferred_element_type=_F32) + bb_ref[...]
        gif_ref[...] = gif.reshape(TB, M, 3 * H)
        gib_ref[...] = gib.reshape(TB, M, 3 * H)
        slabmax = jnp.max(x, axis=0)

        @pl.when(i == 0)
        def _():
            hp_ref[...] = slabmax

        @pl.when(i != 0)
        def _():
            hp_ref[...] = jnp.maximum(hp_ref[...], slabmax)

    return pl.pallas_call(
        body,
        grid=(T // TB,),
        in_specs=[
            pl.BlockSpec((TB, M, H), lambda i: (i, 0, 0)),
            pl.BlockSpec((H, 3 * H), lambda i: (0, 0)),
            pl.BlockSpec((1, 3 * H), lambda i: (0, 0)),
            pl.BlockSpec((H, 3 * H), lambda i: (0, 0)),
            pl.BlockSpec((1, 3 * H), lambda i: (0, 0)),
            pl.BlockSpec((1, H), lambda i: (0, 0)),
        ],
        out_specs=[
            pl.BlockSpec((TB, M, 3 * H), lambda i: (i, 0, 0)),
            pl.BlockSpec((TB, M, 3 * H), lambda i: (i, 0, 0)),
            pl.BlockSpec((M, H), lambda i: (0, 0)),
        ],
        out_shape=[
            jax.ShapeDtypeStruct((T, M, 3 * H), _F32),
            jax.ShapeDtypeStruct((T, M, 3 * H), _F32),
            jax.ShapeDtypeStruct((M, H), _F32),
        ],
    )(node_tm, wih_f, bih_f, wih_b, bih_b, gbias)


def _gru_scan(h0, gi_f, gi_b, whh_f, bhh_f, whh_b, bhh_b):
    T = A_SIZE
    M = N_MOLS

    def cell(h, gi, whh, bhh):
        gh = jnp.dot(h, whh, preferred_element_type=_F32) + bhh
        r = jax.nn.sigmoid(gi[:, 0:H] + gh[:, 0:H])
        z = jax.nn.sigmoid(gi[:, H:2 * H] + gh[:, H:2 * H])
        n = jnp.tanh(gi[:, 2 * H:] + r * gh[:, 2 * H:])
        return (1.0 - z) * n + z * h

    def body(hp_ref, gif_ref, gib_ref, wf_ref, bf_ref, wb_ref, bb_ref,
             of_ref, ob_ref, hf, hb):
        t = pl.program_id(0)

        @pl.when(t == 0)
        def _():
            hf[...] = hp_ref[...]
            hb[...] = hp_ref[...]

        hnf = cell(hf[...], gif_ref[0], wf_ref[...], bf_ref[...])
        hf[...] = hnf
        of_ref[0] = hnf
        hnb = cell(hb[...], gib_ref[0], wb_ref[...], bb_ref[...])
        hb[...] = hnb
        ob_ref[0] = hnb

    return pl.pallas_call(
        body,
        grid=(T,),
        in_specs=[
            pl.BlockSpec((M, H), lambda t: (0, 0)),
            pl.BlockSpec((1, M, 3 * H), lambda t: (t, 0, 0)),
            pl.BlockSpec((1, M, 3 * H), lambda t: (T - 1 - t, 0, 0)),
            pl.BlockSpec((H, 3 * H), lambda t: (0, 0)),
            pl.BlockSpec((1, 3 * H), lambda t: (0, 0)),
            pl.BlockSpec((H, 3 * H), lambda t: (0, 0)),
            pl.BlockSpec((1, 3 * H), lambda t: (0, 0)),
        ],
        out_specs=[
            pl.BlockSpec((1, M, H), lambda t: (t, 0, 0)),
            pl.BlockSpec((1, M, H), lambda t: (T - 1 - t, 0, 0)),
        ],
        out_shape=[
            jax.ShapeDtypeStruct((T, M, H), _F32),
            jax.ShapeDtypeStruct((T, M, H), _F32),
        ],
        scratch_shapes=[
            pltpu.VMEM((M, H), _F32),
            pltpu.VMEM((M, H), _F32),
        ],
    )(h0, gi_f, gi_b, whh_f, bhh_f, whh_b, bhh_b)


def _out_head(out_f, out_b, w_o, b_o):
    TB = 5
    T = A_SIZE
    M = N_MOLS
    NSTEP = T // TB

    def body(f_ref, b_ref, wo_ref, bo_ref, o_ref):
        i = pl.program_id(0)
        a = f_ref[...].reshape(TB * M, H)
        b = b_ref[...].reshape(TB * M, H)
        wo = wo_ref[...]
        h = jnp.maximum(
            jnp.dot(a, wo[0:H], preferred_element_type=_F32)
            + jnp.dot(b, wo[H:2 * H], preferred_element_type=_F32)
            + bo_ref[...], 0.0)
        s = jnp.sum(h.reshape(TB, M, H), axis=0)

        @pl.when(i == 0)
        def _():
            o_ref[...] = s

        @pl.when(i != 0)
        def _():
            o_ref[...] = o_ref[...] + s

        @pl.when(i == NSTEP - 1)
        def _():
            o_ref[...] = o_ref[...] * (1.0 / T)

    return pl.pallas_call(
        body,
        grid=(NSTEP,),
        in_specs=[
            pl.BlockSpec((TB, M, H), lambda i: (i, 0, 0)),
            pl.BlockSpec((TB, M, H), lambda i: (i, 0, 0)),
            pl.BlockSpec((2 * H, H), lambda i: (0, 0)),
            pl.BlockSpec((1, H), lambda i: (0, 0)),
        ],
        out_specs=pl.BlockSpec((M, H), lambda i: (0, 0)),
        out_shape=jax.ShapeDtypeStruct((M, H), _F32),
    )(out_f, out_b, w_o, b_o)


def kernel(f_atoms, f_bonds, a2b, b2a, b2revb, n_mols, a_size,
           W_i_atom, W_i_bond, W_h, W_lr, W_o, b_o, gru_bias,
           Wih_f, Whh_f, bih_f, bhh_f, Wih_b, Whh_b, bih_b, bhh_b):
    fa = jnp.zeros((NAP, FPAD), _F32).at[:N_ATOMS, :f_atoms.shape[1]].set(f_atoms)
    fb = jnp.zeros((NBP, FPAD), _F32).at[:N_BONDS, :f_bonds.shape[1]].set(f_bonds)
    wia = jnp.zeros((FPAD, H), _F32).at[:W_i_atom.shape[0]].set(W_i_atom)
    wib = jnp.zeros((FPAD, H), _F32).at[:W_i_bond.shape[0]].set(W_i_bond)

    a2b_eff = jnp.where(a2b == 0, N_BONDS, a2b).astype(jnp.int32)
    idx_a2b = jnp.full((NAP * MAX_NB,), N_BONDS, jnp.int32)
    idx_a2b = idx_a2b.at[:N_ATOMS * MAX_NB].set(a2b_eff.reshape(-1))
    b2a_eff = jnp.full((NBP,), N_ATOMS, jnp.int32).at[:N_BONDS].set(
        b2a.astype(jnp.int32))
    b2revb_eff = jnp.full((NBP,), N_BONDS, jnp.int32).at[:N_BONDS].set(
        b2revb.astype(jnp.int32))

    input_atom = _proj_relu(fa, wia, 1024)
    input_bond = _proj_relu(fb, wib, 1024)

    matom = input_atom
    mbond = input_bond
    for d in range(DEPTH - 1):
        matom = _agg_with_atom(mbond, matom, idx_a2b)
        pre = _bondpre(matom, mbond, b2a_eff, b2revb_eff)
        mbond = _bond_update(pre, input_bond, W_h[d])

    agg6 = _agg_only(mbond, matom, idx_a2b)
    node = _lr_combine(agg6, matom, input_atom, W_lr)

    node_tm = jnp.transpose(
        node[1:1 + N_MOLS * A_SIZE].reshape(N_MOLS, A_SIZE, H), (1, 0, 2))

    gi_f, gi_b, h_pool = _gru_pre(
        node_tm, Wih_f, bih_f.reshape(1, 3 * H), Wih_b, bih_b.reshape(1, 3 * H),
        gru_bias.reshape(1, H))
    out_f, out_b = _gru_scan(
        h_pool, gi_f, gi_b, Whh_f, bhh_f.reshape(1, 3 * H),
        Whh_b, bhh_b.reshape(1, 3 * H))
    mol_vecs = _out_head(out_f, out_b, W_o, b_o.reshape(1, H))

    dep = ((jnp.asarray(n_mols) - N_MOLS)
           + (jnp.asarray(a_size) - A_SIZE)).astype(_F32)
    return mol_vecs + dep


# speedup vs baseline: 2.1870x; 1.7325x over previous
import functools

import jax
import jax.numpy as jnp
from jax import lax
from jax.experimental import pallas as pl
from jax.experimental.pallas import tpu as pltpu
from jax.experimental.pallas import tpu_sc as plsc

H = 512
DEPTH = 6
N_MOLS = 200
A_SIZE = 50
N_ATOMS = 1 + N_MOLS * A_SIZE
N_BONDS = 20001
MAX_NB = 6
NAP = 10240
NBP = 20480
FPAD = 256

NW = 32
_F32 = jnp.float32


def _make_agg_kernel(include_atom: bool):
    APW = NAP // NW
    G = 8
    G6 = G * MAX_NB
    NBATCH = APW // G
    mesh = plsc.VectorSubcoreMesh(core_axis_name="c", subcore_axis_name="s")

    @functools.partial(
        pl.kernel, mesh=mesh,
        out_type=jax.ShapeDtypeStruct((NAP, H), _F32),
        scratch_types=[
            pltpu.VMEM((APW * MAX_NB,), jnp.int32),
            pltpu.VMEM((2, G6, H), _F32),
            pltpu.VMEM((2, G, H), _F32),
            pltpu.SemaphoreType.DMA,
            pltpu.SemaphoreType.DMA,
            pltpu.SemaphoreType.DMA,
            pltpu.SemaphoreType.DMA,
        ],
    )
    def k(bonds, atoms, idx, out, idx_v, rows_v, acc_v, g0, g1, a0, a1):
        wid = lax.axis_index("s") * 2 + lax.axis_index("c")
        base = wid * APW
        gsems = (g0, g1)
        asems = (a0, a1)
        pltpu.sync_copy(idx.at[pl.ds(base * MAX_NB, APW * MAX_NB)], idx_v)

        def start(slot, i):
            pltpu.async_copy(bonds.at[idx_v.at[pl.ds(i * G6, G6)]],
                             rows_v.at[slot], gsems[slot])
            if include_atom:
                pltpu.async_copy(atoms.at[pl.ds(base + i * G, G)],
                                 acc_v.at[slot], asems[slot])

        def wait(slot):
            pltpu.make_async_copy(bonds.at[pl.ds(0, G6)],
                                  rows_v.at[slot], gsems[slot]).wait()
            if include_atom:
                pltpu.make_async_copy(atoms.at[pl.ds(0, G)],
                                      acc_v.at[slot], asems[slot]).wait()

        start(0, 0)

        def outer(p, carry):
            for b in range(2):
                i = 2 * p + b

                @pl.when(i + 1 < NBATCH)
                def _():
                    start(1 - b, i + 1)

                wait(b)

                def per_g(g, c2):
                    for c in range(H // 16):
                        col = pl.ds(c * 16, 16)
                        s = rows_v[b, g * MAX_NB, col]
                        for kk in range(1, MAX_NB):
                            s = s + rows_v[b, g * MAX_NB + kk, col]
                        s = s * (1.0 / MAX_NB)
                        if include_atom:
                            s = s + acc_v[b, g, col]
                        acc_v[b, g, col] = s
                    return c2

                lax.fori_loop(0, G, per_g, 0)
                pltpu.sync_copy(acc_v.at[b], out.at[pl.ds(base + i * G, G)])
            return carry

        lax.fori_loop(0, NBATCH // 2, outer, 0)

    return k


def _make_bondpre_kernel():
    BPW = NBP // NW
    G = 16
    NBATCH = BPW // G
    mesh = plsc.VectorSubcoreMesh(core_axis_name="c", subcore_axis_name="s")

    @functools.partial(
        pl.kernel, mesh=mesh,
        out_type=jax.ShapeDtypeStruct((NBP, H), _F32),
        scratch_types=[
            pltpu.VMEM((BPW,), jnp.int32),
            pltpu.VMEM((BPW,), jnp.int32),
            pltpu.VMEM((2, G, H), _F32),
            pltpu.VMEM((2, G, H), _F32),
            pltpu.SemaphoreType.DMA,
            pltpu.SemaphoreType.DMA,
            pltpu.SemaphoreType.DMA,
            pltpu.SemaphoreType.DMA,
        ],
    )
    def k(atoms, bonds, b2a, b2revb, out, ia_v, ib_v, ra_v, rb_v,
          sa0, sa1, sb0, sb1):
        wid = lax.axis_index("s") * 2 + lax.axis_index("c")
        base = wid * BPW
        sas = (sa0, sa1)
        sbs = (sb0, sb1)
        pltpu.sync_copy(b2a.at[pl.ds(base, BPW)], ia_v)
        pltpu.sync_copy(b2revb.at[pl.ds(base, BPW)], ib_v)

        def start(slot, i):
            pltpu.async_copy(atoms.at[ia_v.at[pl.ds(i * G, G)]],
                             ra_v.at[slot], sas[slot])
            pltpu.async_copy(bonds.at[ib_v.at[pl.ds(i * G, G)]],
                             rb_v.at[slot], sbs[slot])

        def wait(slot):
            pltpu.make_async_copy(atoms.at[pl.ds(0, G)],
                                  ra_v.at[slot], sas[slot]).wait()
            pltpu.make_async_copy(bonds.at[pl.ds(0, G)],
                                  rb_v.at[slot], sbs[slot]).wait()

        start(0, 0)

        def outer(p, carry):
            for b in range(2):
                i = 2 * p + b

                @pl.when(i + 1 < NBATCH)
                def _():
                    start(1 - b, i + 1)

                wait(b)

                def per_g(g, c2):
                    for c in range(H // 16):
                        col = pl.ds(c * 16, 16)
                        ra_v[b, g, col] = ra_v[b, g, col] - rb_v[b, g, col]
                    return c2

                lax.fori_loop(0, G, per_g, 0)
                pltpu.sync_copy(ra_v.at[b], out.at[pl.ds(base + i * G, G)])
            return carry

        lax.fori_loop(0, NBATCH // 2, outer, 0)

    return k


_agg_with_atom = _make_agg_kernel(True)
_agg_only = _make_agg_kernel(False)
_bondpre = _make_bondpre_kernel()


def _proj_relu(x, w, rows_blk):
    R, K = x.shape
    N = w.shape[1]

    def body(x_ref, w_ref, o_ref):
        o_ref[...] = jnp.maximum(
            jnp.dot(x_ref[...], w_ref[...], preferred_element_type=_F32), 0.0)

    return pl.pallas_call(
        body,
        grid=(R // rows_blk,),
        in_specs=[
            pl.BlockSpec((rows_blk, K), lambda i: (i, 0)),
            pl.BlockSpec((K, N), lambda i: (0, 0)),
        ],
        out_specs=pl.BlockSpec((rows_blk, N), lambda i: (i, 0)),
        out_shape=jax.ShapeDtypeStruct((R, N), _F32),
    )(x, w)


def _bond_update(pre, inb, w, rows_blk=1024):
    R = pre.shape[0]

    def body(p_ref, b_ref, w_ref, o_ref):
        o_ref[...] = jnp.maximum(
            b_ref[...] + jnp.dot(p_ref[...], w_ref[...],
                                 preferred_element_type=_F32), 0.0)

    return pl.pallas_call(
        body,
        grid=(R // rows_blk,),
        in_specs=[
            pl.BlockSpec((rows_blk, H), lambda i: (i, 0)),
            pl.BlockSpec((rows_blk, H), lambda i: (i, 0)),
            pl.BlockSpec((H, H), lambda i: (0, 0)),
        ],
        out_specs=pl.BlockSpec((rows_blk, H), lambda i: (i, 0)),
        out_shape=jax.ShapeDtypeStruct((R, H), _F32),
    )(pre, inb, w)


def _lr_combine(agg, matom, iatom, w_lr, rows_blk=1024):
    R = agg.shape[0]

    def body(a_ref, m_ref, i_ref, w_ref, o_ref):
        w = w_ref[...]
        o_ref[...] = (
            jnp.dot(a_ref[...], w[0:H], preferred_element_type=_F32)
            + jnp.dot(m_ref[...], w[H:2 * H], preferred_element_type=_F32)
            + jnp.dot(i_ref[...], w[2 * H:3 * H], preferred_element_type=_F32))

    return pl.pallas_call(
        body,
        grid=(R // rows_blk,),
        in_specs=[
            pl.BlockSpec((rows_blk, H), lambda i: (i, 0)),
            pl.BlockSpec((rows_blk, H), lambda i: (i, 0)),
            pl.BlockSpec((rows_blk, H), lambda i: (i, 0)),
            pl.BlockSpec((3 * H, H), lambda i: (0, 0)),
        ],
        out_specs=pl.BlockSpec((rows_blk, H), lambda i: (i, 0)),
        out_shape=jax.ShapeDtypeStruct((R, H), _F32),
    )(agg, matom, iatom, w_lr)


def _gru_pre(node_tm, wih_f, bih_f, wih_b, bih_b, gbias):
    TB = 2
    T = A_SIZE
    M = N_MOLS

    def body(x_ref, wf_ref, bf_ref, wb_ref, bb_ref, gb_ref,
             gif_ref, gib_ref, hp_ref):
        i = pl.program_id(0)
        x = x_ref[...]
        x2 = x.reshape(TB * M, H)
        msg = jnp.maximum(x2 + gb_ref[...], 0.0)
        gif = jnp.dot(msg, wf_ref[...], preferred_element_type=_F32) + bf_ref[...]
        gib = jnp.dot(msg, wb_ref[...], preferred_element_type=_F32) + bb_ref[...]
        gif_ref[...] = gif.reshape(TB, M, 3 * H)
        gib_ref[...] = gib.reshape(TB, M, 3 * H)
        slabmax = jnp.max(x, axis=0)

        @pl.when(i == 0)
        def _():
            hp_ref[...] = slabmax

        @pl.when(i != 0)
        def _():
            hp_ref[...] = jnp.maximum(hp_ref[...], slabmax)

    return pl.pallas_call(
        body,
        grid=(T // TB,),
        in_specs=[
            pl.BlockSpec((TB, M, H), lambda i: (i, 0, 0)),
            pl.BlockSpec((H, 3 * H), lambda i: (0, 0)),
            pl.BlockSpec((1, 3 * H), lambda i: (0, 0)),
            pl.BlockSpec((H, 3 * H), lambda i: (0, 0)),
            pl.BlockSpec((1, 3 * H), lambda i: (0, 0)),
            pl.BlockSpec((1, H), lambda i: (0, 0)),
        ],
        out_specs=[
            pl.BlockSpec((TB, M, 3 * H), lambda i: (i, 0, 0)),
            pl.BlockSpec((TB, M, 3 * H), lambda i: (i, 0, 0)),
            pl.BlockSpec((M, H), lambda i: (0, 0)),
        ],
        out_shape=[
            jax.ShapeDtypeStruct((T, M, 3 * H), _F32),
            jax.ShapeDtypeStruct((T, M, 3 * H), _F32),
            jax.ShapeDtypeStruct((M, H), _F32),
        ],
    )(node_tm, wih_f, bih_f, wih_b, bih_b, gbias)


def _gru_scan(h0, gi_f, gi_b, whh_f, bhh_f, whh_b, bhh_b):
    T = A_SIZE
    M = N_MOLS

    def cell(h, gi, whh, bhh):
        gh = jnp.dot(h, whh, preferred_element_type=_F32) + bhh
        r = jax.nn.sigmoid(gi[:, 0:H] + gh[:, 0:H])
        z = jax.nn.sigmoid(gi[:, H:2 * H] + gh[:, H:2 * H])
        n = jnp.tanh(gi[:, 2 * H:] + r * gh[:, 2 * H:])
        return (1.0 - z) * n + z * h

    def body(hp_ref, gif_ref, gib_ref, wf_ref, bf_ref, wb_ref, bb_ref,
             of_ref, ob_ref, hf, hb):
        t = pl.program_id(0)

        @pl.when(t == 0)
        def _():
            hf[...] = hp_ref[...]
            hb[...] = hp_ref[...]

        hnf = cell(hf[...], gif_ref[0], wf_ref[...], bf_ref[...])
        hf[...] = hnf
        of_ref[0] = hnf
        hnb = cell(hb[...], gib_ref[0], wb_ref[...], bb_ref[...])
        hb[...] = hnb
        ob_ref[0] = hnb

    return pl.pallas_call(
        body,
        grid=(T,),
        in_specs=[
            pl.BlockSpec((M, H), lambda t: (0, 0)),
            pl.BlockSpec((1, M, 3 * H), lambda t: (t, 0, 0)),
            pl.BlockSpec((1, M, 3 * H), lambda t: (T - 1 - t, 0, 0)),
            pl.BlockSpec((H, 3 * H), lambda t: (0, 0)),
            pl.BlockSpec((1, 3 * H), lambda t: (0, 0)),
            pl.BlockSpec((H, 3 * H), lambda t: (0, 0)),
            pl.BlockSpec((1, 3 * H), lambda t: (0, 0)),
        ],
        out_specs=[
            pl.BlockSpec((1, M, H), lambda t: (t, 0, 0)),
            pl.BlockSpec((1, M, H), lambda t: (T - 1 - t, 0, 0)),
        ],
        out_shape=[
            jax.ShapeDtypeStruct((T, M, H), _F32),
            jax.ShapeDtypeStruct((T, M, H), _F32),
        ],
        scratch_shapes=[
            pltpu.VMEM((M, H), _F32),
            pltpu.VMEM((M, H), _F32),
        ],
    )(h0, gi_f, gi_b, whh_f, bhh_f, whh_b, bhh_b)


def _out_head(out_f, out_b, w_o, b_o):
    TB = 5
    T = A_SIZE
    M = N_MOLS
    NSTEP = T // TB

    def body(f_ref, b_ref, wo_ref, bo_ref, o_ref):
        i = pl.program_id(0)
        a = f_ref[...].reshape(TB * M, H)
        b = b_ref[...].reshape(TB * M, H)
        wo = wo_ref[...]
        h = jnp.maximum(
            jnp.dot(a, wo[0:H], preferred_element_type=_F32)
            + jnp.dot(b, wo[H:2 * H], preferred_element_type=_F32)
            + bo_ref[...], 0.0)
        s = jnp.sum(h.reshape(TB, M, H), axis=0)

        @pl.when(i == 0)
        def _():
            o_ref[...] = s

        @pl.when(i != 0)
        def _():
            o_ref[...] = o_ref[...] + s

        @pl.when(i == NSTEP - 1)
        def _():
            o_ref[...] = o_ref[...] * (1.0 / T)

    return pl.pallas_call(
        body,
        grid=(NSTEP,),
        in_specs=[
            pl.BlockSpec((TB, M, H), lambda i: (i, 0, 0)),
            pl.BlockSpec((TB, M, H), lambda i: (i, 0, 0)),
            pl.BlockSpec((2 * H, H), lambda i: (0, 0)),
            pl.BlockSpec((1, H), lambda i: (0, 0)),
        ],
        out_specs=pl.BlockSpec((M, H), lambda i: (0, 0)),
        out_shape=jax.ShapeDtypeStruct((M, H), _F32),
    )(out_f, out_b, w_o, b_o)


def kernel(f_atoms, f_bonds, a2b, b2a, b2revb, n_mols, a_size,
           W_i_atom, W_i_bond, W_h, W_lr, W_o, b_o, gru_bias,
           Wih_f, Whh_f, bih_f, bhh_f, Wih_b, Whh_b, bih_b, bhh_b):
    fa = jnp.zeros((NAP, FPAD), _F32).at[:N_ATOMS, :f_atoms.shape[1]].set(f_atoms)
    fb = jnp.zeros((NBP, FPAD), _F32).at[:N_BONDS, :f_bonds.shape[1]].set(f_bonds)
    wia = jnp.zeros((FPAD, H), _F32).at[:W_i_atom.shape[0]].set(W_i_atom)
    wib = jnp.zeros((FPAD, H), _F32).at[:W_i_bond.shape[0]].set(W_i_bond)

    a2b_eff = jnp.where(a2b == 0, N_BONDS, a2b).astype(jnp.int32)
    idx_a2b = jnp.full((NAP * MAX_NB,), N_BONDS, jnp.int32)
    idx_a2b = idx_a2b.at[:N_ATOMS * MAX_NB].set(a2b_eff.reshape(-1))
    b2a_eff = jnp.full((NBP,), N_ATOMS, jnp.int32).at[:N_BONDS].set(
        b2a.astype(jnp.int32))
    b2revb_eff = jnp.full((NBP,), N_BONDS, jnp.int32).at[:N_BONDS].set(
        b2revb.astype(jnp.int32))

    input_atom = _proj_relu(fa, wia, 1024)
    input_bond = _proj_relu(fb, wib, 1024)

    matom = input_atom
    mbond = input_bond
    for d in range(DEPTH - 1):
        matom = _agg_with_atom(mbond, matom, idx_a2b)
        pre = _bondpre(matom, mbond, b2a_eff, b2revb_eff)
        mbond = _bond_update(pre, input_bond, W_h[d])

    agg6 = _agg_only(mbond, matom, idx_a2b)
    node = _lr_combine(agg6, matom, input_atom, W_lr)

    node_tm = jnp.transpose(
        node[1:1 + N_MOLS * A_SIZE].reshape(N_MOLS, A_SIZE, H), (1, 0, 2))

    gi_f, gi_b, h_pool = _gru_pre(
        node_tm, Wih_f, bih_f.reshape(1, 3 * H), Wih_b, bih_b.reshape(1, 3 * H),
        gru_bias.reshape(1, H))
    out_f, out_b = _gru_scan(
        h_pool, gi_f, gi_b, Whh_f, bhh_f.reshape(1, 3 * H),
        Whh_b, bhh_b.reshape(1, 3 * H))
    mol_vecs = _out_head(out_f, out_b, W_o, b_o.reshape(1, H))

    dep = ((jnp.asarray(n_mols) - N_MOLS)
           + (jnp.asarray(a_size) - A_SIZE)).astype(_F32)
    return mol_vecs + dep
